# Initial kernel scaffold; baseline (speedup 1.0000x reference)
import jax
import jax.numpy as jnp
from jax.experimental import pallas as pl


def kernel(x, edge_index, edge_attr, We1, be1, We2, be2, Wn1, bn1, Wn2, bn2):
    raise NotImplementedError("write your pallas kernel here")


# speedup vs baseline: 2.6001x; 2.6001x over previous
import functools

import jax
import jax.numpy as jnp
from jax import lax
from jax.experimental import pallas as pl
from jax.experimental.pallas import tpu as pltpu
from jax.experimental.pallas import tpu_sc as plsc

N = 10000
E = 320000
D = 128
DE = 16
H = 128

NC = 2
NS = 16
NW = NC * NS

CHUNK = 128
ROWS = E // CHUNK
ROWS_PER_W = ROWS // NW
ROWS_LEFT = ROWS - ROWS_PER_W * NW
ACC_W = 128
NPS = N // NS

@functools.lru_cache(maxsize=1)
def _mesh():
    return plsc.VectorSubcoreMesh(core_axis_name="c", subcore_axis_name="s",
                                  num_cores=NC, num_subcores=NS)


def _proj_body(x_ref, w1r_ref, w1c_ref, be1_ref, pr_ref, pc_ref):
    xb = x_ref[...]
    pr_ref[...] = jnp.dot(xb, w1r_ref[...], preferred_element_type=jnp.float32) + be1_ref[...]
    pc_ref[...] = jnp.dot(xb, w1c_ref[...], preferred_element_type=jnp.float32)


def _node_proj(x, w1r, w1c, be1):
    nb = 10
    bn = N // nb
    return pl.pallas_call(
        _proj_body,
        grid=(nb,),
        in_specs=[
            pl.BlockSpec((bn, D), lambda i: (i, 0)),
            pl.BlockSpec((D, H), lambda i: (0, 0)),
            pl.BlockSpec((D, H), lambda i: (0, 0)),
            pl.BlockSpec((1, H), lambda i: (0, 0)),
        ],
        out_specs=[
            pl.BlockSpec((bn, H), lambda i: (i, 0)),
            pl.BlockSpec((bn, H), lambda i: (i, 0)),
        ],
        out_shape=[
            jax.ShapeDtypeStruct((N, H), jnp.float32),
            jax.ShapeDtypeStruct((N, H), jnp.float32),
        ],
    )(x, w1r, w1c, be1)


def _gather_add_kernel(pr_hbm, pc_hbm, row2_hbm, col2_hbm, s_hbm,
                       idxr, idxc, buf_r, buf_c, sem_r, sem_c):
    c = lax.axis_index("c")
    s = lax.axis_index("s")
    wid = s * NC + c
    row0 = wid * ROWS_PER_W

    def chunk(r):
        base = r * CHUNK
        pltpu.sync_copy(row2_hbm.at[pl.ds(r, 1)], idxr)
        pltpu.sync_copy(col2_hbm.at[pl.ds(r, 1)], idxc)
        cp_r = pltpu.async_copy(pr_hbm.at[idxr.at[0]], buf_r, sem_r)
        cp_c = pltpu.async_copy(pc_hbm.at[idxc.at[0]], buf_c, sem_c)
        cp_r.wait()
        cp_c.wait()

        def add_row(j, _):
            for i in range(H // 16):
                sl = pl.ds(i * 16, 16)
                buf_r[j, sl] = buf_r[j, sl] + buf_c[j, sl]
            return 0

        lax.fori_loop(0, CHUNK, add_row, 0)
        pltpu.sync_copy(buf_r, s_hbm.at[pl.ds(base, CHUNK)])

    def body(k, _):
        chunk(row0 + k)
        return 0

    lax.fori_loop(0, ROWS_PER_W, body, 0)

    @pl.when(wid < ROWS_LEFT)
    def _():
        chunk(ROWS_PER_W * NW + wid)


def _gather_add(pr, pc, row2, col2):
    return pl.kernel(
        _gather_add_kernel,
        out_type=jax.ShapeDtypeStruct((E, H), jnp.float32),
        mesh=_mesh(),
        scratch_types=[
            pltpu.VMEM((1, CHUNK), jnp.int32),
            pltpu.VMEM((1, CHUNK), jnp.int32),
            pltpu.VMEM((CHUNK, H), jnp.float32),
            pltpu.VMEM((CHUNK, H), jnp.float32),
            pltpu.SemaphoreType.DMA,
            pltpu.SemaphoreType.DMA,
        ],
    )(pr, pc, row2, col2)


def _edge_mlp_body(s_ref, ea_ref, w1a_ref, w2_ref, be2_ref, out_ref):
    ea = ea_ref[...]
    h1 = jax.nn.silu(s_ref[...] + jnp.dot(ea, w1a_ref[...], preferred_element_type=jnp.float32))
    h2 = jax.nn.silu(jnp.dot(h1, w2_ref[...], preferred_element_type=jnp.float32) + be2_ref[...])
    out_ref[...] = h2 + ea


def _edge_mlp(s, ea, w1a, w2, be2):
    be = 512
    nb = E // be
    return pl.pallas_call(
        _edge_mlp_body,
        grid=(nb,),
        in_specs=[
            pl.BlockSpec((be, H), lambda i: (i, 0)),
            pl.BlockSpec((be, DE), lambda i: (i, 0)),
            pl.BlockSpec((DE, H), lambda i: (0, 0)),
            pl.BlockSpec((H, DE), lambda i: (0, 0)),
            pl.BlockSpec((1, DE), lambda i: (0, 0)),
        ],
        out_specs=pl.BlockSpec((be, DE), lambda i: (i, 0)),
        out_shape=jax.ShapeDtypeStruct((E, DE), jnp.float32),
    )(s, ea, w1a, w2, be2)


def _scatter_kernel(eof_hbm, col2_hbm, zeros_hbm, tmpl_hbm, out_hbm,
                    idxc, fbuf, buf128, acc):
    c = lax.axis_index("c")
    s = lax.axis_index("s")
    wid = s * NC + c
    row0 = wid * ROWS_PER_W

    pltpu.sync_copy(zeros_hbm, acc.at[pl.ds(s * NPS, NPS)])

    pltpu.sync_copy(tmpl_hbm, buf128)

    plsc.subcore_barrier()

    def chunk(r):
        pltpu.sync_copy(col2_hbm.at[pl.ds(r, 1)], idxc)
        pltpu.sync_copy(eof_hbm.at[pl.ds(r * CHUNK * DE, CHUNK * DE)], fbuf)

        def mv(j, _):
            buf128[j, pl.ds(0, DE)] = fbuf[pl.ds(j * DE, DE)]
            return 0

        lax.fori_loop(0, CHUNK, mv, 0)
        pltpu.sync_copy(buf128, acc.at[idxc.at[0]], add=True)

    def body(k, _):
        chunk(row0 + k)
        return 0

    lax.fori_loop(0, ROWS_PER_W, body, 0)

    @pl.when(wid < ROWS_LEFT)
    def _():
        chunk(ROWS_PER_W * NW + wid)

    plsc.subcore_barrier()

    pltpu.sync_copy(acc.at[pl.ds(s * NPS, NPS)], out_hbm.at[c, s])


def _scatter_mean_sum(eo, col2):
    zeros = jnp.zeros((NPS, ACC_W), jnp.float32)
    tmpl = jnp.zeros((CHUNK, ACC_W), jnp.float32).at[:, DE].set(1.0)
    return pl.kernel(
        _scatter_kernel,
        out_type=jax.ShapeDtypeStruct((NC, NS, NPS, ACC_W), jnp.float32),
        mesh=_mesh(),
        scratch_types=[
            pltpu.VMEM((1, CHUNK), jnp.int32),
            pltpu.VMEM((CHUNK * DE,), jnp.float32),
            pltpu.VMEM((CHUNK, ACC_W), jnp.float32),
            pltpu.VMEM_SHARED((N, ACC_W), jnp.float32),
        ],
    )(eo.reshape(E * DE), col2, zeros, tmpl)


def _node_mlp_body(p0_ref, p1_ref, x_ref, wn1a_ref, wn1b_ref, bn1_ref,
                   wn2_ref, bn2_ref, out_ref):
    ps = p0_ref[...] + p1_ref[...]
    cnt = jnp.maximum(ps[:, DE:DE + 1], 1.0)
    aggr = ps[:, :DE] / cnt
    xb = x_ref[...]
    g1 = jax.nn.silu(
        jnp.dot(aggr, wn1a_ref[...], preferred_element_type=jnp.float32)
        + jnp.dot(xb, wn1b_ref[...], preferred_element_type=jnp.float32)
        + bn1_ref[...])
    g2 = jax.nn.silu(jnp.dot(g1, wn2_ref[...], preferred_element_type=jnp.float32) + bn2_ref[...])
    out_ref[...] = g2 + xb


def _node_mlp(p0, p1, x, wn1a, wn1b, bn1, wn2, bn2):
    nb = 10
    bn = N // nb
    return pl.pallas_call(
        _node_mlp_body,
        grid=(nb,),
        in_specs=[
            pl.BlockSpec((bn, ACC_W), lambda i: (i, 0)),
            pl.BlockSpec((bn, ACC_W), lambda i: (i, 0)),
            pl.BlockSpec((bn, D), lambda i: (i, 0)),
            pl.BlockSpec((DE, H), lambda i: (0, 0)),
            pl.BlockSpec((D, H), lambda i: (0, 0)),
            pl.BlockSpec((1, H), lambda i: (0, 0)),
            pl.BlockSpec((H, D), lambda i: (0, 0)),
            pl.BlockSpec((1, D), lambda i: (0, 0)),
        ],
        out_specs=pl.BlockSpec((bn, D), lambda i: (i, 0)),
        out_shape=jax.ShapeDtypeStruct((N, D), jnp.float32),
    )(p0, p1, x, wn1a, wn1b, bn1, wn2, bn2)


@jax.jit
def kernel(x, edge_index, edge_attr, We1, be1, We2, be2, Wn1, bn1, Wn2, bn2):
    row2 = edge_index[0].reshape(ROWS, CHUNK)
    col2 = edge_index[1].reshape(ROWS, CHUNK)

    w1a = We1[:DE]
    w1r = We1[DE:DE + D]
    w1c = We1[DE + D:]

    pr, pc = _node_proj(x, w1r, w1c, be1.reshape(1, H))
    s = _gather_add(pr, pc, row2, col2)
    edge_out = _edge_mlp(s, edge_attr, w1a, We2, be2.reshape(1, DE))
    partials = _scatter_mean_sum(edge_out, col2).reshape(NC, N, ACC_W)
    x_out = _node_mlp(partials[0], partials[1], x,
                      Wn1[:DE], Wn1[DE:], bn1.reshape(1, H),
                      Wn2, bn2.reshape(1, D))
    return (x_out, edge_out)


# speedup vs baseline: 3.3237x; 1.2783x over previous
import functools

import jax
import jax.numpy as jnp
from jax import lax
from jax.experimental import pallas as pl
from jax.experimental.pallas import tpu as pltpu
from jax.experimental.pallas import tpu_sc as plsc

N = 10000
E = 320000
D = 128
DE = 16
H = 128

NC = 2
NS = 16
NW = NC * NS

CHUNK = 128
ROWS = E // CHUNK
RPW = 80
ROWSP = NW * RPW
ACC_W = 128
NPS = N // NS

@functools.lru_cache(maxsize=1)
def _mesh():
    return plsc.VectorSubcoreMesh(core_axis_name="c", subcore_axis_name="s",
                                  num_cores=NC, num_subcores=NS)


def _proj_body(x_ref, w1r_ref, w1c_ref, be1_ref, pr_ref, pc_ref):
    xb = x_ref[...]
    pr_ref[...] = jnp.dot(xb, w1r_ref[...], preferred_element_type=jnp.float32) + be1_ref[...]
    pc_ref[...] = jnp.dot(xb, w1c_ref[...], preferred_element_type=jnp.float32)


def _node_proj(x, w1r, w1c, be1):
    nb = 10
    bn = N // nb
    return pl.pallas_call(
        _proj_body,
        grid=(nb,),
        in_specs=[
            pl.BlockSpec((bn, D), lambda i: (i, 0)),
            pl.BlockSpec((D, H), lambda i: (0, 0)),
            pl.BlockSpec((D, H), lambda i: (0, 0)),
            pl.BlockSpec((1, H), lambda i: (0, 0)),
        ],
        out_specs=[
            pl.BlockSpec((bn, H), lambda i: (i, 0)),
            pl.BlockSpec((bn, H), lambda i: (i, 0)),
        ],
        out_shape=[
            jax.ShapeDtypeStruct((N, H), jnp.float32),
            jax.ShapeDtypeStruct((N, H), jnp.float32),
        ],
    )(x, w1r, w1c, be1)


def _gather_add_kernel(pr_hbm, pc_hbm, row2_hbm, col2_hbm, s_hbm,
                       idxr, idxc, br0, br1, bc0, bc1,
                       sr0, sr1, sc0, sc1, sw0, sw1):
    c = lax.axis_index("c")
    s = lax.axis_index("s")
    wid = s * NC + c
    row0 = wid * RPW

    brs, bcs = (br0, br1), (bc0, bc1)
    srs, scs, sws = (sr0, sr1), (sc0, sc1), (sw0, sw1)

    pltpu.sync_copy(row2_hbm.at[pl.ds(row0, RPW)], idxr)
    pltpu.sync_copy(col2_hbm.at[pl.ds(row0, RPW)], idxc)

    def valid(k):
        return row0 + k < ROWS

    def fire(k, b):
        @pl.when(valid(k))
        def _():
            pltpu.async_copy(pr_hbm.at[idxr.at[k]], brs[b], srs[b])
            pltpu.async_copy(pc_hbm.at[idxc.at[k]], bcs[b], scs[b])

    def addw(k, b):
        @pl.when(valid(k))
        def _():
            pltpu.make_async_copy(pr_hbm.at[idxr.at[k]], brs[b], srs[b]).wait()
            pltpu.make_async_copy(pc_hbm.at[idxc.at[k]], bcs[b], scs[b]).wait()

            def add_row(j, _):
                for i in range(H // 16):
                    sl = pl.ds(i * 16, 16)
                    brs[b][j, sl] = brs[b][j, sl] + bcs[b][j, sl]
                return 0

            lax.fori_loop(0, CHUNK, add_row, 0)
            base = (row0 + k) * CHUNK
            pltpu.async_copy(brs[b], s_hbm.at[pl.ds(base, CHUNK)], sws[b])

    def wait_w(k, b):
        @pl.when(valid(k))
        def _():
            base = (row0 + k) * CHUNK
            pltpu.make_async_copy(brs[b], s_hbm.at[pl.ds(base, CHUNK)], sws[b]).wait()

    fire(0, 0)

    def outer(kk, _):
        for b in (0, 1):
            k = 2 * kk + b

            @pl.when(k + 1 < RPW)
            def _():
                pl.when(k >= 1)(lambda: wait_w(k - 1, 1 - b))
                fire(k + 1, 1 - b)

            addw(k, b)
        return 0

    lax.fori_loop(0, RPW // 2, outer, 0)
    wait_w(RPW - 2, 0)
    wait_w(RPW - 1, 1)


def _gather_add(pr, pc, row2, col2):
    return pl.kernel(
        _gather_add_kernel,
        out_type=jax.ShapeDtypeStruct((E, H), jnp.float32),
        mesh=_mesh(),
        scratch_types=[
            pltpu.VMEM((RPW, CHUNK), jnp.int32),
            pltpu.VMEM((RPW, CHUNK), jnp.int32),
            pltpu.VMEM((CHUNK, H), jnp.float32),
            pltpu.VMEM((CHUNK, H), jnp.float32),
            pltpu.VMEM((CHUNK, H), jnp.float32),
            pltpu.VMEM((CHUNK, H), jnp.float32),
            pltpu.SemaphoreType.DMA,
            pltpu.SemaphoreType.DMA,
            pltpu.SemaphoreType.DMA,
            pltpu.SemaphoreType.DMA,
            pltpu.SemaphoreType.DMA,
            pltpu.SemaphoreType.DMA,
        ],
    )(pr, pc, row2, col2)


def _edge_mlp_body(s_ref, ea_ref, w1a_ref, w2_ref, be2_ref, out_ref):
    ea = ea_ref[...]
    h1 = jax.nn.silu(s_ref[...] + jnp.dot(ea, w1a_ref[...], preferred_element_type=jnp.float32))
    h2 = jax.nn.silu(jnp.dot(h1, w2_ref[...], preferred_element_type=jnp.float32) + be2_ref[...])
    out_ref[...] = h2 + ea


def _edge_mlp(s, ea, w1a, w2, be2):
    be = 512
    nb = E // be
    return pl.pallas_call(
        _edge_mlp_body,
        grid=(nb,),
        in_specs=[
            pl.BlockSpec((be, H), lambda i: (i, 0)),
            pl.BlockSpec((be, DE), lambda i: (i, 0)),
            pl.BlockSpec((DE, H), lambda i: (0, 0)),
            pl.BlockSpec((H, DE), lambda i: (0, 0)),
            pl.BlockSpec((1, DE), lambda i: (0, 0)),
        ],
        out_specs=pl.BlockSpec((be, DE), lambda i: (i, 0)),
        out_shape=jax.ShapeDtypeStruct((E, DE), jnp.float32),
    )(s, ea, w1a, w2, be2)


def _scatter_kernel(eof_hbm, col2_hbm, zeros_hbm, tmpl_hbm, out_hbm,
                    idxc, f0, f1, m0, m1, acc, sf0, sf1, ss0, ss1):
    c = lax.axis_index("c")
    s = lax.axis_index("s")
    wid = s * NC + c
    row0 = wid * RPW

    fbufs, mbufs = (f0, f1), (m0, m1)
    sfs, sss = (sf0, sf1), (ss0, ss1)

    pltpu.sync_copy(zeros_hbm, acc.at[pl.ds(s * NPS, NPS)])

    pltpu.sync_copy(tmpl_hbm, m0)
    pltpu.sync_copy(tmpl_hbm, m1)
    pltpu.sync_copy(col2_hbm.at[pl.ds(row0, RPW)], idxc)

    plsc.subcore_barrier()

    def valid(k):
        return row0 + k < ROWS

    def fire(k, b):
        @pl.when(valid(k))
        def _():
            base = (row0 + k) * CHUNK * DE
            pltpu.async_copy(eof_hbm.at[pl.ds(base, CHUNK * DE)], fbufs[b], sfs[b])

    def proc(k, b):
        @pl.when(valid(k))
        def _():
            base = (row0 + k) * CHUNK * DE
            pltpu.make_async_copy(eof_hbm.at[pl.ds(base, CHUNK * DE)], fbufs[b], sfs[b]).wait()

            def mv(j, _):
                mbufs[b][j, pl.ds(0, DE)] = fbufs[b][pl.ds(j * DE, DE)]
                return 0

            lax.fori_loop(0, CHUNK, mv, 0)
            pltpu.async_copy(mbufs[b], acc.at[idxc.at[k]], sss[b], add=True)

    def wait_s(k, b):
        @pl.when(valid(k))
        def _():
            pltpu.make_async_copy(mbufs[b], acc.at[idxc.at[k]], sss[b]).wait()

    fire(0, 0)

    def outer(kk, _):
        for b in (0, 1):
            k = 2 * kk + b

            @pl.when(k + 1 < RPW)
            def _():
                fire(k + 1, 1 - b)

            pl.when(k >= 2)(lambda: wait_s(k - 2, b))
            proc(k, b)
        return 0

    lax.fori_loop(0, RPW // 2, outer, 0)
    wait_s(RPW - 2, 0)
    wait_s(RPW - 1, 1)

    plsc.subcore_barrier()

    pltpu.sync_copy(acc.at[pl.ds(s * NPS, NPS)], out_hbm.at[c, s])


def _scatter_mean_sum(eo, col2):
    zeros = jnp.zeros((NPS, ACC_W), jnp.float32)
    tmpl = jnp.zeros((CHUNK, ACC_W), jnp.float32).at[:, DE].set(1.0)
    return pl.kernel(
        _scatter_kernel,
        out_type=jax.ShapeDtypeStruct((NC, NS, NPS, ACC_W), jnp.float32),
        mesh=_mesh(),
        scratch_types=[
            pltpu.VMEM((RPW, CHUNK), jnp.int32),
            pltpu.VMEM((CHUNK * DE,), jnp.float32),
            pltpu.VMEM((CHUNK * DE,), jnp.float32),
            pltpu.VMEM((CHUNK, ACC_W), jnp.float32),
            pltpu.VMEM((CHUNK, ACC_W), jnp.float32),
            pltpu.VMEM_SHARED((N, ACC_W), jnp.float32),
            pltpu.SemaphoreType.DMA,
            pltpu.SemaphoreType.DMA,
            pltpu.SemaphoreType.DMA,
            pltpu.SemaphoreType.DMA,
        ],
    )(eo.reshape(E * DE), col2, zeros, tmpl)


def _node_mlp_body(p0_ref, p1_ref, x_ref, wn1a_ref, wn1b_ref, bn1_ref,
                   wn2_ref, bn2_ref, out_ref):
    ps = p0_ref[...] + p1_ref[...]
    cnt = jnp.maximum(ps[:, DE:DE + 1], 1.0)
    aggr = ps[:, :DE] / cnt
    xb = x_ref[...]
    g1 = jax.nn.silu(
        jnp.dot(aggr, wn1a_ref[...], preferred_element_type=jnp.float32)
        + jnp.dot(xb, wn1b_ref[...], preferred_element_type=jnp.float32)
        + bn1_ref[...])
    g2 = jax.nn.silu(jnp.dot(g1, wn2_ref[...], preferred_element_type=jnp.float32) + bn2_ref[...])
    out_ref[...] = g2 + xb


def _node_mlp(p0, p1, x, wn1a, wn1b, bn1, wn2, bn2):
    nb = 10
    bn = N // nb
    return pl.pallas_call(
        _node_mlp_body,
        grid=(nb,),
        in_specs=[
            pl.BlockSpec((bn, ACC_W), lambda i: (i, 0)),
            pl.BlockSpec((bn, ACC_W), lambda i: (i, 0)),
            pl.BlockSpec((bn, D), lambda i: (i, 0)),
            pl.BlockSpec((DE, H), lambda i: (0, 0)),
            pl.BlockSpec((D, H), lambda i: (0, 0)),
            pl.BlockSpec((1, H), lambda i: (0, 0)),
            pl.BlockSpec((H, D), lambda i: (0, 0)),
            pl.BlockSpec((1, D), lambda i: (0, 0)),
        ],
        out_specs=pl.BlockSpec((bn, D), lambda i: (i, 0)),
        out_shape=jax.ShapeDtypeStruct((N, D), jnp.float32),
    )(p0, p1, x, wn1a, wn1b, bn1, wn2, bn2)


@jax.jit
def kernel(x, edge_index, edge_attr, We1, be1, We2, be2, Wn1, bn1, Wn2, bn2):
    pad = ((0, ROWSP - ROWS), (0, 0))
    row2 = jnp.pad(edge_index[0].reshape(ROWS, CHUNK), pad)
    col2 = jnp.pad(edge_index[1].reshape(ROWS, CHUNK), pad)

    w1a = We1[:DE]
    w1r = We1[DE:DE + D]
    w1c = We1[DE + D:]

    pr, pc = _node_proj(x, w1r, w1c, be1.reshape(1, H))
    s = _gather_add(pr, pc, row2, col2)
    edge_out = _edge_mlp(s, edge_attr, w1a, We2, be2.reshape(1, DE))
    partials = _scatter_mean_sum(edge_out, col2).reshape(NC, N, ACC_W)
    x_out = _node_mlp(partials[0], partials[1], x,
                      Wn1[:DE], Wn1[DE:], bn1.reshape(1, H),
                      Wn2, bn2.reshape(1, D))
    return (x_out, edge_out)
